# baseline (speedup 1.0000x reference)
import functools

import numpy as np

import jax
import jax.numpy as jnp
from jax import lax
from jax.experimental import pallas as pl
from jax.experimental.pallas import tpu as pltpu
from jax.experimental.pallas import tpu_sc as plsc

_B = 4
_S = 48
_R = _B * _S
_H = 256
_NZ = 96
_ROWS = 2 * _B * _S * _S * 2 * 2 // 2 + 1
_STAGE_ROWS = _NZ * 4 * _R

_NW = 32
_CHUNK = 64
_NCH = 18
_PW = _NCH * _CHUNK
_TAIL = 8
_BPAD = _NW * _PW + _TAIL


def _block_count(u):
    if u < 0 or u > 2 * _S - 2:
        return 0
    return (u + 1) if u < _S else (2 * _S - 1 - u)


def _perm_table():
    zero_row = 95 * 4 * _R + _R
    perm = np.full((_BPAD,), zero_row, np.int32)
    wp = 1
    for z in range(_NZ):
        for d in (0, 1):
            u = z - d
            a = _block_count(u)
            if a == 0:
                continue
            lo = max(0, u - (_S - 1))
            for xy in (0, 1):
                blk = z * 4 * _R + (2 * d + xy) * _R
                for n in range(_B):
                    for i in range(lo, lo + a):
                        perm[wp] = blk + n * _S + i
                        wp += 1
    assert wp == _ROWS
    return perm


_PERM = _perm_table()


_UNROLL = 8


def _shift_raw(val):
    return jnp.concatenate(
        [jnp.zeros((1, _H), jnp.float32), val[:-1, :]], axis=0)


def _shift_seg(val, i_idx):
    return jnp.where(i_idx == 0, 0.0, _shift_raw(val))


def _dotf(a, b):
    return jnp.dot(a, b, preferred_element_type=jnp.float32)


def _tc_body(src_ref, trg_ref, wxx_ref, wxy_ref, whcat_ref,
             bx_ref, by_ref, out_ref, hx0, hy0, hx1, hy1, ytbuf, srcx,
             tyt):
    g = pl.program_id(0)

    @pl.when(g == 0)
    def _init():
        for r in (hx0, hy0, hx1, hy1, ytbuf):
            r[...] = jnp.zeros((_R, _H), jnp.float32)
        srcx[...] = _dotf(src_ref[...], wxx_ref[0])
        tyt[...] = jnp.zeros((8 * _S, _H), jnp.float32)
        ty = _dotf(trg_ref[...], wxy_ref[0])
        for zz in range(_S):
            for n in range(_B):
                tyt[zz * 8 + n:zz * 8 + n + 1, :] = (
                    ty[n * _S + zz:n * _S + zz + 1, :])

    i_idx = jax.lax.rem(
        jax.lax.broadcasted_iota(jnp.int32, (_R, 1), 0), _S)

    for sub in range(_UNROLL):
        z = g * _UNROLL + sub
        base = sub * 4 * _R

        u1 = z - 1
        lo1 = jnp.maximum(u1 - (_S - 1), 0)
        hi1 = jnp.minimum(u1, _S - 1)
        hx1s = _shift_seg(hx1[...], i_idx)
        hterm1 = _dotf(hx1s, whcat_ref[1, :_H, :]) + _dotf(
            hy1[...], whcat_ref[1, _H:, :])
        ox1 = jnp.tanh(_dotf(hx0[...], wxx_ref[1])
                       + hterm1[:, :_H] + bx_ref[1:2, :])
        oy1 = jnp.tanh(_dotf(hy0[...], wxy_ref[1])
                       + hterm1[:, _H:] + by_ref[1:2, :])
        mask1 = (i_idx >= lo1) & (i_idx <= hi1)
        x1 = ox1
        y1 = jnp.where(mask1, oy1, 0.0)
        hx1[...] = x1
        hy1[...] = y1
        out_ref[base + 2 * _R:base + 3 * _R, :] = x1
        out_ref[base + 3 * _R:base + 4 * _R, :] = y1

        u0 = z
        lo0 = jnp.maximum(u0 - (_S - 1), 0)
        hi0 = jnp.minimum(u0, _S - 1)

        ytbuf[...] = _shift_raw(ytbuf[...])
        tz = tyt[pl.ds(jnp.minimum(u0, _S - 1) * 8, 8)]
        tz = jnp.where(u0 < _S, tz, 0.0)
        for n in range(_B):
            ytbuf[n * _S:n * _S + 1, :] = tz[n:n + 1, :]

        hx0s = _shift_seg(hx0[...], i_idx)
        hterm0 = _dotf(hx0s, whcat_ref[0, :_H, :]) + _dotf(
            hy0[...], whcat_ref[0, _H:, :])
        ox0 = jnp.tanh(srcx[...] + hterm0[:, :_H] + bx_ref[0:1, :])
        oy0 = jnp.tanh(ytbuf[...] + hterm0[:, _H:] + by_ref[0:1, :])
        mask0 = (i_idx >= lo0) & (i_idx <= hi0)
        x0 = ox0
        y0 = jnp.where(mask0, oy0, 0.0)
        hx0[...] = x0
        hy0[...] = y0
        out_ref[base:base + _R, :] = x0
        out_ref[base + _R:base + 2 * _R, :] = y0


def _forward_staging(src, trg, Wx_x, Wh_x, b_x, Wx_y, Wh_y, b_y):
    whcat = jnp.concatenate([Wh_x, Wh_y], axis=2)

    full = lambda shp: pl.BlockSpec(shp, lambda z: (0,) * len(shp))
    return pl.pallas_call(
        _tc_body,
        grid=(_NZ // _UNROLL,),
        in_specs=[
            full((_R, _H)),
            full((_R, _H)),
            full((2, _H, _H)),
            full((2, _H, _H)),
            full((2, 2 * _H, 2 * _H)),
            full((2, _H)),
            full((2, _H)),
        ],
        out_specs=pl.BlockSpec((_UNROLL * 4 * _R, _H), lambda g: (g, 0)),
        out_shape=jax.ShapeDtypeStruct((_STAGE_ROWS, _H), jnp.float32),
        scratch_shapes=[pltpu.VMEM((_R, _H), jnp.float32)] * 6
        + [pltpu.VMEM((8 * _S, _H), jnp.float32)],
        compiler_params=pltpu.CompilerParams(
            dimension_semantics=("arbitrary",),
        ),
    )(src, trg, Wx_x, Wx_y, whcat, b_x, b_y)


def _sc_body(table_hbm, idx_hbm, out_hbm, idx_v, r0, r1, r2, r3,
             tail_i, tail_r, gsem, osem, tsem):
    wid = lax.axis_index("s") * 2 + lax.axis_index("c")
    base = wid * _PW
    pltpu.sync_copy(idx_hbm.at[pl.ds(base, _PW)], idx_v)
    bufs = (r0, r1, r2, r3)

    def _gather(c):
        return pltpu.async_copy(
            table_hbm.at[idx_v.at[pl.ds(c * _CHUNK, _CHUNK)]],
            bufs[c % 4], gsem)

    outs = []
    gathers = [_gather(0), _gather(1)]
    for c in range(_NCH):
        if c + 2 < _NCH:
            if c + 2 >= 4:
                outs[c - 2].wait()
            gathers.append(_gather(c + 2))
        gathers[c].wait()
        o = pltpu.make_async_copy(
            bufs[c % 4], out_hbm.at[pl.ds(base + c * _CHUNK, _CHUNK)], osem)
        o.start()
        outs.append(o)

    @pl.when(wid == 0)
    def _tail():
        pltpu.sync_copy(idx_hbm.at[pl.ds(_NW * _PW, _TAIL)], tail_i)
        pltpu.async_copy(table_hbm.at[tail_i], tail_r, tsem).wait()
        pltpu.sync_copy(tail_r, out_hbm.at[pl.ds(_NW * _PW, _TAIL)])

    for o in outs[-4:]:
        o.wait()


def _sc_permute(staging, idx):
    mesh = plsc.VectorSubcoreMesh(core_axis_name="c", subcore_axis_name="s")
    k = functools.partial(
        pl.kernel,
        mesh=mesh,
        out_type=jax.ShapeDtypeStruct((_BPAD, _H), jnp.float32),
        scratch_types=[
            pltpu.VMEM((_PW,), jnp.int32),
            pltpu.VMEM((_CHUNK, _H), jnp.float32),
            pltpu.VMEM((_CHUNK, _H), jnp.float32),
            pltpu.VMEM((_CHUNK, _H), jnp.float32),
            pltpu.VMEM((_CHUNK, _H), jnp.float32),
            pltpu.VMEM((_TAIL,), jnp.int32),
            pltpu.VMEM((_TAIL, _H), jnp.float32),
            pltpu.SemaphoreType.DMA,
            pltpu.SemaphoreType.DMA,
            pltpu.SemaphoreType.DMA,
        ],
    )(_sc_body)
    return k(staging, idx)


def kernel(src_array_batch, trg_array_batch, src_lens, trg_lens,
           Wx_x, Wh_x, b_x, Wx_y, Wh_y, b_y):
    del src_lens, trg_lens
    staging = _forward_staging(src_array_batch, trg_array_batch,
                               Wx_x, Wh_x, b_x, Wx_y, Wh_y, b_y)
    out = _sc_permute(staging, jnp.asarray(_PERM))
    return out[:_ROWS]


# speedup vs baseline: 1.0061x; 1.0061x over previous
import functools

import numpy as np

import jax
import jax.numpy as jnp
from jax import lax
from jax.experimental import pallas as pl
from jax.experimental.pallas import tpu as pltpu
from jax.experimental.pallas import tpu_sc as plsc

_B = 4
_S = 48
_R = _B * _S
_H = 256
_NZ = 96
_ROWS = 2 * _B * _S * _S * 2 * 2 // 2 + 1
_STAGE_ROWS = _NZ * 4 * _R

_NW = 32
_CHUNK = 128
_NCH = 9
_PW = _NCH * _CHUNK
_TAIL = 8
_BPAD = _NW * _PW + _TAIL


def _block_count(u):
    if u < 0 or u > 2 * _S - 2:
        return 0
    return (u + 1) if u < _S else (2 * _S - 1 - u)


def _perm_table():
    zero_row = 95 * 4 * _R + _R
    perm = np.full((_BPAD,), zero_row, np.int32)
    wp = 1
    for z in range(_NZ):
        for d in (0, 1):
            u = z - d
            a = _block_count(u)
            if a == 0:
                continue
            lo = max(0, u - (_S - 1))
            for xy in (0, 1):
                blk = z * 4 * _R + (2 * d + xy) * _R
                for n in range(_B):
                    for i in range(lo, lo + a):
                        perm[wp] = blk + n * _S + i
                        wp += 1
    assert wp == _ROWS
    return perm


_PERM = _perm_table()


_UNROLL = 8


def _shift_raw(val):
    return jnp.concatenate(
        [jnp.zeros((1, _H), jnp.float32), val[:-1, :]], axis=0)


def _shift_seg(val, i_idx):
    return jnp.where(i_idx == 0, 0.0, _shift_raw(val))


def _dotf(a, b):
    return jnp.dot(a, b, preferred_element_type=jnp.float32)


def _tc_body(src_ref, trg_ref, wxx_ref, wxy_ref, whcat_ref,
             bx_ref, by_ref, out_ref, hx0, hy0, hx1, hy1, ytbuf, srcx,
             tyt):
    g = pl.program_id(0)

    @pl.when(g == 0)
    def _init():
        for r in (hx0, hy0, hx1, hy1, ytbuf):
            r[...] = jnp.zeros((_R, _H), jnp.float32)
        srcx[...] = _dotf(src_ref[...], wxx_ref[0])
        tyt[...] = jnp.zeros((8 * _S, _H), jnp.float32)
        ty = _dotf(trg_ref[...], wxy_ref[0])
        for zz in range(_S):
            for n in range(_B):
                tyt[zz * 8 + n:zz * 8 + n + 1, :] = (
                    ty[n * _S + zz:n * _S + zz + 1, :])

    i_idx = jax.lax.rem(
        jax.lax.broadcasted_iota(jnp.int32, (_R, 1), 0), _S)

    for sub in range(_UNROLL):
        z = g * _UNROLL + sub
        base = sub * 4 * _R

        u1 = z - 1
        lo1 = jnp.maximum(u1 - (_S - 1), 0)
        hi1 = jnp.minimum(u1, _S - 1)
        hx1s = _shift_seg(hx1[...], i_idx)
        hterm1 = _dotf(hx1s, whcat_ref[1, :_H, :]) + _dotf(
            hy1[...], whcat_ref[1, _H:, :])
        ox1 = jnp.tanh(_dotf(hx0[...], wxx_ref[1])
                       + hterm1[:, :_H] + bx_ref[1:2, :])
        oy1 = jnp.tanh(_dotf(hy0[...], wxy_ref[1])
                       + hterm1[:, _H:] + by_ref[1:2, :])
        mask1 = (i_idx >= lo1) & (i_idx <= hi1)
        x1 = ox1
        y1 = jnp.where(mask1, oy1, 0.0)
        hx1[...] = x1
        hy1[...] = y1
        out_ref[base + 2 * _R:base + 3 * _R, :] = x1
        out_ref[base + 3 * _R:base + 4 * _R, :] = y1

        u0 = z
        lo0 = jnp.maximum(u0 - (_S - 1), 0)
        hi0 = jnp.minimum(u0, _S - 1)

        ytbuf[...] = _shift_raw(ytbuf[...])
        tz = tyt[pl.ds(jnp.minimum(u0, _S - 1) * 8, 8)]
        tz = jnp.where(u0 < _S, tz, 0.0)
        for n in range(_B):
            ytbuf[n * _S:n * _S + 1, :] = tz[n:n + 1, :]

        hx0s = _shift_seg(hx0[...], i_idx)
        hterm0 = _dotf(hx0s, whcat_ref[0, :_H, :]) + _dotf(
            hy0[...], whcat_ref[0, _H:, :])
        ox0 = jnp.tanh(srcx[...] + hterm0[:, :_H] + bx_ref[0:1, :])
        oy0 = jnp.tanh(ytbuf[...] + hterm0[:, _H:] + by_ref[0:1, :])
        mask0 = (i_idx >= lo0) & (i_idx <= hi0)
        x0 = ox0
        y0 = jnp.where(mask0, oy0, 0.0)
        hx0[...] = x0
        hy0[...] = y0
        out_ref[base:base + _R, :] = x0
        out_ref[base + _R:base + 2 * _R, :] = y0


def _forward_staging(src, trg, Wx_x, Wh_x, b_x, Wx_y, Wh_y, b_y):
    whcat = jnp.concatenate([Wh_x, Wh_y], axis=2)

    full = lambda shp: pl.BlockSpec(shp, lambda z: (0,) * len(shp))
    return pl.pallas_call(
        _tc_body,
        grid=(_NZ // _UNROLL,),
        in_specs=[
            full((_R, _H)),
            full((_R, _H)),
            full((2, _H, _H)),
            full((2, _H, _H)),
            full((2, 2 * _H, 2 * _H)),
            full((2, _H)),
            full((2, _H)),
        ],
        out_specs=pl.BlockSpec((_UNROLL * 4 * _R, _H), lambda g: (g, 0)),
        out_shape=jax.ShapeDtypeStruct((_STAGE_ROWS, _H), jnp.float32),
        scratch_shapes=[pltpu.VMEM((_R, _H), jnp.float32)] * 6
        + [pltpu.VMEM((8 * _S, _H), jnp.float32)],
        compiler_params=pltpu.CompilerParams(
            dimension_semantics=("arbitrary",),
        ),
    )(src, trg, Wx_x, Wx_y, whcat, b_x, b_y)


def _sc_body(table_hbm, idx_hbm, out_hbm, idx_v, r0, r1, r2,
             tail_i, tail_r, gsem, osem, tsem):
    wid = lax.axis_index("s") * 2 + lax.axis_index("c")
    base = wid * _PW
    pltpu.sync_copy(idx_hbm.at[pl.ds(base, _PW)], idx_v)
    bufs = (r0, r1, r2)

    def _gather(c):
        return pltpu.async_copy(
            table_hbm.at[idx_v.at[pl.ds(c * _CHUNK, _CHUNK)]],
            bufs[c % 3], gsem)

    outs = []
    gathers = [_gather(0)]
    for c in range(_NCH):
        if c + 1 < _NCH:
            if c + 1 >= 3:
                outs[c - 2].wait()
            gathers.append(_gather(c + 1))
        gathers[c].wait()
        o = pltpu.make_async_copy(
            bufs[c % 3], out_hbm.at[pl.ds(base + c * _CHUNK, _CHUNK)], osem)
        o.start()
        outs.append(o)

    @pl.when(wid == 0)
    def _tail():
        pltpu.sync_copy(idx_hbm.at[pl.ds(_NW * _PW, _TAIL)], tail_i)
        pltpu.async_copy(table_hbm.at[tail_i], tail_r, tsem).wait()
        pltpu.sync_copy(tail_r, out_hbm.at[pl.ds(_NW * _PW, _TAIL)])

    for o in outs[-3:]:
        o.wait()


def _sc_permute(staging, idx):
    mesh = plsc.VectorSubcoreMesh(core_axis_name="c", subcore_axis_name="s")
    k = functools.partial(
        pl.kernel,
        mesh=mesh,
        out_type=jax.ShapeDtypeStruct((_BPAD, _H), jnp.float32),
        scratch_types=[
            pltpu.VMEM((_PW,), jnp.int32),
            pltpu.VMEM((_CHUNK, _H), jnp.float32),
            pltpu.VMEM((_CHUNK, _H), jnp.float32),
            pltpu.VMEM((_CHUNK, _H), jnp.float32),
            pltpu.VMEM((_TAIL,), jnp.int32),
            pltpu.VMEM((_TAIL, _H), jnp.float32),
            pltpu.SemaphoreType.DMA,
            pltpu.SemaphoreType.DMA,
            pltpu.SemaphoreType.DMA,
        ],
    )(_sc_body)
    return k(staging, idx)


def kernel(src_array_batch, trg_array_batch, src_lens, trg_lens,
           Wx_x, Wh_x, b_x, Wx_y, Wh_y, b_y):
    del src_lens, trg_lens
    staging = _forward_staging(src_array_batch, trg_array_batch,
                               Wx_x, Wh_x, b_x, Wx_y, Wh_y, b_y)
    out = _sc_permute(staging, jnp.asarray(_PERM))
    return out[:_ROWS]
